# baseline (speedup 1.0000x reference)
import functools

import jax
import jax.numpy as jnp
from jax import lax
from jax.experimental import pallas as pl
from jax.experimental.pallas import tpu as pltpu
from jax.experimental.pallas import tpu_sc as plsc

B = 16384
D = 32
L = 16
NC = 2
NS = 16
NW = NC * NS
BPW = B // NW
GROUPS = BPW // L
NCHUNK = BPW // 128

_mesh = plsc.VectorSubcoreMesh(core_axis_name="c", subcore_axis_name="s")


@functools.partial(
    pl.kernel,
    mesh=_mesh,
    out_type=jax.ShapeDtypeStruct((B,), jnp.float32),
    scratch_types=[
        pltpu.VMEM((NCHUNK, 128), jnp.int32),
        pltpu.VMEM((NCHUNK, 128), jnp.int32),
        pltpu.VMEM((BPW, D), jnp.float32),
        pltpu.VMEM((BPW, D), jnp.float32),
        pltpu.VMEM((D, L), jnp.float32),
        pltpu.VMEM((L,), jnp.float32),
        pltpu.VMEM((BPW,), jnp.float32),
        pltpu.SemaphoreType.DMA,
        pltpu.SemaphoreType.DMA,
    ],
    compiler_params=pltpu.CompilerParams(
        needs_layout_passes=False, use_tc_tiling_on_sc=False),
)
def _cf_kernel(uid_hbm, iid_hbm, ut_hbm, it_hbm, wsp_hbm, b_hbm, out_hbm,
               uid_v, iid_v, urows, irows, wsp_v, b_v, out_v, usem, isem):
    wid = lax.axis_index("s") * NC + lax.axis_index("c")
    base_row = wid * NCHUNK

    pltpu.sync_copy(uid_hbm.at[pl.ds(base_row, NCHUNK)], uid_v)
    pltpu.sync_copy(iid_hbm.at[pl.ds(base_row, NCHUNK)], iid_v)

    copies = []
    for j in range(NCHUNK):
        copies.append(pltpu.async_copy(
            ut_hbm.at[uid_v.at[j]], urows.at[pl.ds(j * 128, 128)], usem))
        copies.append(pltpu.async_copy(
            it_hbm.at[iid_v.at[j]], irows.at[pl.ds(j * 128, 128)], isem))
    pltpu.sync_copy(wsp_hbm, wsp_v)
    pltpu.sync_copy(b_hbm, b_v)
    for c in copies:
        c.wait()

    bias = b_v[...]
    lane = lax.iota(jnp.int32, L)
    wvecs = [wsp_v[d] for d in range(D)]
    cols = [jnp.full((L,), d, jnp.int32) for d in range(D)]

    def group_body(g, carry):
        row = g * L + lane
        acc = bias
        for d in range(D):
            u = plsc.load_gather(urows, [row, cols[d]])
            v = plsc.load_gather(irows, [row, cols[d]])
            acc = acc + u * v * wvecs[d]
        out_v[pl.ds(g * L, L)] = acc
        return carry

    lax.fori_loop(0, GROUPS, group_body, 0)

    pltpu.sync_copy(out_v, out_hbm.at[pl.ds(wid * BPW, BPW)])


def kernel(user_id, item_id, user_table, item_table, fc_w, fc_b):
    uid2 = user_id.reshape(B // 128, 128)
    iid2 = item_id.reshape(B // 128, 128)
    wsp = jnp.broadcast_to(fc_w.reshape(D, 1), (D, L))
    b16 = jnp.broadcast_to(fc_b.reshape(1), (L,)).astype(jnp.float32)
    return _cf_kernel(uid2, iid2, user_table, item_table, wsp, b16)
